# baseline (speedup 1.0000x reference)
import functools

import jax
import jax.numpy as jnp
from jax import lax
from jax.experimental import pallas as pl
from jax.experimental.pallas import tpu as pltpu
from jax.experimental.pallas import tpu_sc as plsc

BATCH = 16384
EMB = 64
NC = 2
NS = 16
NW = NC * NS
B_PER_W = BATCH // NW
CHUNK = 128
NCHUNK = B_PER_W // CHUNK


_SHUF_DNUMS = lax.GatherDimensionNumbers(
    offset_dims=(), collapsed_slice_dims=(0,), start_index_map=(0,))


def _shuffle(x, perm):
    return lax.gather(x, perm[:, None], _SHUF_DNUMS, slice_sizes=(1,),
                      mode=lax.GatherScatterMode.PROMISE_IN_BOUNDS)


def _body(u_hbm, v_hbm, user_hbm, item_hbm, out_hbm,
          idx_u, idx_v, ue, ve, out_v, sem):
    wid = lax.axis_index("s") * NC + lax.axis_index("c")
    base = wid * B_PER_W

    for j in range(NCHUNK):
        pltpu.sync_copy(u_hbm.at[pl.ds(base + j * CHUNK, CHUNK)], idx_u.at[j])
        pltpu.sync_copy(v_hbm.at[pl.ds(base + j * CHUNK, CHUNK)], idx_v.at[j])

    copies = []
    for j in range(NCHUNK):
        dst_rows = pl.ds(j * CHUNK, CHUNK)
        copies.append(pltpu.async_copy(user_hbm.at[idx_u.at[j]], ue.at[dst_rows], sem))
        copies.append(pltpu.async_copy(item_hbm.at[idx_v.at[j]], ve.at[dst_rows], sem))
    for c in copies:
        c.wait()

    lanes = lax.iota(jnp.int32, 16)
    zero16 = jnp.zeros((16,), jnp.float32)

    def group_body(g, carry):
        r0 = g * 16
        acc = zero16
        for k in range(16):
            r = r0 + k
            p = ue[r, pl.ds(0, 16)] * ve[r, pl.ds(0, 16)]
            for q in range(1, EMB // 16):
                p = p + ue[r, pl.ds(q * 16, 16)] * ve[r, pl.ds(q * 16, 16)]
            for s in (8, 4, 2, 1):
                p = p + _shuffle(p, lanes ^ s)
            acc = jnp.where(lanes == k, p, acc)
        out_v[pl.ds(r0, 16)] = acc
        return carry

    lax.fori_loop(0, B_PER_W // 16, group_body, 0)

    pltpu.sync_copy(out_v, out_hbm.at[pl.ds(base, B_PER_W)])


@jax.jit
def _run(u, v, user_emb, item_emb):
    mesh = plsc.VectorSubcoreMesh(core_axis_name="c", subcore_axis_name="s")
    kfn = functools.partial(
        pl.kernel,
        mesh=mesh,
        compiler_params=pltpu.CompilerParams(use_tc_tiling_on_sc=False),
        out_type=jax.ShapeDtypeStruct((BATCH,), jnp.float32),
        scratch_types=[
            pltpu.VMEM((NCHUNK, CHUNK), jnp.int32),
            pltpu.VMEM((NCHUNK, CHUNK), jnp.int32),
            pltpu.VMEM((B_PER_W, EMB), jnp.float32),
            pltpu.VMEM((B_PER_W, EMB), jnp.float32),
            pltpu.VMEM((B_PER_W,), jnp.float32),
            pltpu.SemaphoreType.DMA,
        ],
    )(_body)
    return kfn(u, v, user_emb, item_emb)


def kernel(u, v, user_emb, item_emb):
    return _run(u, v, user_emb, item_emb)
